# Initial kernel scaffold; baseline (speedup 1.0000x reference)
import jax
import jax.numpy as jnp
from jax.experimental import pallas as pl


def kernel(emb1, emb2, W1, b1, W2, b2):
    raise NotImplementedError("write your pallas kernel here")


# speedup vs baseline: 30.9906x; 30.9906x over previous
import functools

import jax
import jax.numpy as jnp
from jax import lax
from jax.experimental import pallas as pl
from jax.experimental.pallas import tpu as pltpu
from jax.experimental.pallas import tpu_sc as plsc

N = 10000
DIM = 128
NUM_EDGES = 320000
A1 = 0.1
A2 = 2.0

NC = 2
NS = 16
LANES = 16
NWORKERS = NC * NS

TOTAL = N * N
WIN = 20000
NWIN = TOTAL // WIN
NBINS = 2048


def _nodevec_body(emb1_ref, emb2_ref, w1_ref, b1_ref, w2_ref, b2_ref,
                  nv1_ref, nv2_ref):
    dn = (((1,), (1,)), ((), ()))
    z1 = lax.dot_general(emb1_ref[...], w1_ref[...], dn,
                         preferred_element_type=jnp.float32)
    nv1_ref[...] = jnp.tanh(A1 * (z1 + b1_ref[...]))
    z2 = lax.dot_general(emb2_ref[...], w2_ref[...], dn,
                         preferred_element_type=jnp.float32)
    nv2_ref[...] = jnp.tanh(A1 * (z2 + b2_ref[...]))


def _nodevecs(emb1, emb2, W1, b1, W2, b2):
    br = 1000
    grid = (N // br,)
    return pl.pallas_call(
        _nodevec_body,
        grid=grid,
        in_specs=[
            pl.BlockSpec((br, DIM), lambda i: (i, 0)),
            pl.BlockSpec((br, DIM), lambda i: (i, 0)),
            pl.BlockSpec((DIM, DIM), lambda i: (0, 0)),
            pl.BlockSpec((1, DIM), lambda i: (0, 0)),
            pl.BlockSpec((DIM, DIM), lambda i: (0, 0)),
            pl.BlockSpec((1, DIM), lambda i: (0, 0)),
        ],
        out_specs=[
            pl.BlockSpec((br, DIM), lambda i: (i, 0)),
            pl.BlockSpec((br, DIM), lambda i: (i, 0)),
        ],
        out_shape=[
            jax.ShapeDtypeStruct((N, DIM), jnp.float32),
            jax.ShapeDtypeStruct((N, DIM), jnp.float32),
        ],
    )(emb1, emb2, W1, b1.reshape(1, DIM), W2, b2.reshape(1, DIM))


def _score_body(nv1_ref, nv2_ref, out_ref):
    dn = (((1,), (1,)), ((), ()))
    u = lax.dot_general(nv1_ref[...], nv2_ref[...], dn,
                        preferred_element_type=jnp.float32)
    out_ref[...] = 1.0 / (1.0 + jnp.exp(-A2 * u))


def _scores(nv1, nv2):
    br, bc = 256, 2048
    grid = (N // br + (N % br != 0), N // bc + (N % bc != 0))
    return pl.pallas_call(
        _score_body,
        grid=grid,
        in_specs=[
            pl.BlockSpec((br, DIM), lambda i, j: (i, 0)),
            pl.BlockSpec((bc, DIM), lambda i, j: (j, 0)),
        ],
        out_specs=pl.BlockSpec((br, bc), lambda i, j: (i, j)),
        out_shape=jax.ShapeDtypeStruct((N, N), jnp.float32),
    )(nv1, nv2)


def _hist_pass(shift, width, pmask):
    binmask = (1 << width) - 1
    if pmask >= 1 << 31:
        pmask -= 1 << 32
    mesh = plsc.VectorSubcoreMesh(core_axis_name="c", subcore_axis_name="s",
                                  num_cores=NC, num_subcores=NS)
    nextra = NWIN - (NWIN // NWORKERS) * NWORKERS
    nbase = NWIN // NWORKERS

    @functools.partial(
        pl.kernel,
        out_type=jax.ShapeDtypeStruct((NWORKERS * LANES * NBINS,), jnp.int32),
        mesh=mesh,
        compiler_params=pltpu.CompilerParams(needs_layout_passes=False),
        scratch_types=[
            pltpu.VMEM((WIN,), jnp.float32),
            pltpu.VMEM((LANES,), jnp.int32),
            pltpu.VMEM((LANES * NBINS,), jnp.int32),
        ],
    )
    def hist_kernel(s_hbm, pval_hbm, out_hbm, buf, pvalv_ref, hist):
        cid = lax.axis_index("c")
        sid = lax.axis_index("s")
        wid = sid * NC + cid

        def zero_body(i, carry):
            hist[pl.ds(i * LANES, LANES)] = jnp.zeros((LANES,), jnp.int32)
            return carry

        lax.fori_loop(0, (LANES * NBINS) // LANES, zero_body, 0)

        pltpu.sync_copy(pval_hbm, pvalv_ref)
        pval = pvalv_ref[...]
        pm = jnp.full((LANES,), pmask, jnp.int32)
        shv = jnp.full((LANES,), shift, jnp.int32)
        bm = jnp.full((LANES,), binmask, jnp.int32)
        lane_off = lax.iota(jnp.int32, LANES) * NBINS
        ones = jnp.ones((LANES,), jnp.int32)
        nwin_w = nbase + jnp.where(wid < nextra, 1, 0)

        def win_body(g, carry):
            win = wid + g * NWORKERS
            pltpu.sync_copy(s_hbm.at[pl.ds(win * WIN, WIN)], buf)

            def vec_body(j, c2):
                x = buf[pl.ds(j * LANES, LANES)]
                key = plsc.bitcast(x, jnp.int32)
                b = lax.shift_right_logical(key, shv) & bm
                sel = (key & pm) == pval
                plsc.addupdate_scatter(hist, [b + lane_off], ones, mask=sel)
                return c2

            lax.fori_loop(0, WIN // LANES, vec_body, 0)
            return carry

        lax.fori_loop(0, nwin_w, win_body, 0)
        pltpu.sync_copy(
            hist, out_hbm.at[pl.ds(wid * LANES * NBINS, LANES * NBINS)])

    return hist_kernel


_HIST_PASSES = [
    (21, 11, 0x00000000),
    (10, 11, 0xFFE00000),
    (0, 10, 0xFFFFFC00),
]
_HIST_KERNELS = None


def _get_hist_kernels():
    global _HIST_KERNELS
    if _HIST_KERNELS is None:
        _HIST_KERNELS = [_hist_pass(*p) for p in _HIST_PASSES]
    return _HIST_KERNELS


def _select_body(shift, hists_ref, carry_ref, out_ref):
    h = jnp.sum(hists_ref[...], axis=0, keepdims=True)
    iota = lax.broadcasted_iota(jnp.int32, (1, NBINS), 1)
    c = h
    sh = 1
    while sh < NBINS:
        r = pltpu.roll(c, sh, axis=1)
        c = c + jnp.where(iota >= sh, r, 0)
        sh *= 2
    tot = jnp.sum(h, axis=1, keepdims=True)
    cnt_gt = tot - c
    cnt_ge = cnt_gt + h
    krem = carry_ref[1]
    sel = (cnt_gt < krem) & (cnt_ge >= krem)
    bstar = jnp.max(jnp.where(sel, iota, -1), axis=(0, 1))
    cgt = jnp.max(jnp.where(sel, cnt_gt, -1), axis=(0, 1))
    out_ref[0] = carry_ref[0] + bstar * (1 << shift)
    out_ref[1] = krem - cgt
    out_ref[2] = 0
    out_ref[3] = 0


def _select_bin(shift, hists, carry):
    return pl.pallas_call(
        functools.partial(_select_body, shift),
        in_specs=[
            pl.BlockSpec((NWORKERS * LANES, NBINS), lambda: (0, 0)),
            pl.BlockSpec(memory_space=pltpu.SMEM),
        ],
        out_specs=pl.BlockSpec(memory_space=pltpu.SMEM),
        out_shape=jax.ShapeDtypeStruct((4,), jnp.int32),
    )(hists, carry)


def _mask_body(br, bc, s_ref, tk_ref, out_ref):
    i = pl.program_id(0)
    j = pl.program_id(1)
    s = s_ref[...]
    key = lax.bitcast_convert_type(s, jnp.int32)
    kept = jnp.where(key >= tk_ref[0], s, 0.0)
    row = i * br + lax.broadcasted_iota(jnp.int32, (br, bc), 0)
    col = j * bc + lax.broadcasted_iota(jnp.int32, (br, bc), 1)
    out_ref[...] = jnp.where(row == col, jnp.maximum(kept, 0.5), kept)


def _mask(S, tkey):
    br, bc = 256, 2048
    grid = (N // br + (N % br != 0), N // bc + (N % bc != 0))
    return pl.pallas_call(
        functools.partial(_mask_body, br, bc),
        grid=grid,
        in_specs=[
            pl.BlockSpec((br, bc), lambda i, j: (i, j)),
            pl.BlockSpec(memory_space=pltpu.SMEM),
        ],
        out_specs=pl.BlockSpec((br, bc), lambda i, j: (i, j)),
        out_shape=jax.ShapeDtypeStruct((N, N), jnp.float32),
    )(S, tkey)


def kernel(emb1, emb2, W1, b1, W2, b2):
    nv1, nv2 = _nodevecs(emb1, emb2, W1, b1, W2, b2)
    S = _scores(nv1, nv2)
    flat = S.reshape(TOTAL)
    carry = jnp.array([0, NUM_EDGES, 0, 0], jnp.int32)
    for (shift, _w, _m), hk in zip(_HIST_PASSES, _get_hist_kernels()):
        pvalv = jnp.broadcast_to(carry[0], (LANES,)).astype(jnp.int32)
        hists = hk(flat, pvalv)
        carry = _select_bin(shift, hists.reshape(NWORKERS * LANES, NBINS),
                            carry)
    return _mask(S, carry)


# speedup vs baseline: 35.3658x; 1.1412x over previous
import functools

import jax
import jax.numpy as jnp
from jax import lax
from jax.experimental import pallas as pl
from jax.experimental.pallas import tpu as pltpu
from jax.experimental.pallas import tpu_sc as plsc

N = 10000
DIM = 128
NUM_EDGES = 320000
A1 = 0.1
A2 = 2.0

NC = 2
NS = 16
LANES = 16
NWORKERS = NC * NS

TOTAL = N * N
WIN = 16000
NWIN = TOTAL // WIN
NBINS = 2048
NCOPY = 2
UNROLL = 8


def _nodevec_body(emb1_ref, emb2_ref, w1_ref, b1_ref, w2_ref, b2_ref,
                  nv1_ref, nv2_ref):
    dn = (((1,), (1,)), ((), ()))
    z1 = lax.dot_general(emb1_ref[...], w1_ref[...], dn,
                         preferred_element_type=jnp.float32)
    nv1_ref[...] = jnp.tanh(A1 * (z1 + b1_ref[...]))
    z2 = lax.dot_general(emb2_ref[...], w2_ref[...], dn,
                         preferred_element_type=jnp.float32)
    nv2_ref[...] = jnp.tanh(A1 * (z2 + b2_ref[...]))


def _nodevecs(emb1, emb2, W1, b1, W2, b2):
    br = 1000
    grid = (N // br,)
    return pl.pallas_call(
        _nodevec_body,
        grid=grid,
        in_specs=[
            pl.BlockSpec((br, DIM), lambda i: (i, 0)),
            pl.BlockSpec((br, DIM), lambda i: (i, 0)),
            pl.BlockSpec((DIM, DIM), lambda i: (0, 0)),
            pl.BlockSpec((1, DIM), lambda i: (0, 0)),
            pl.BlockSpec((DIM, DIM), lambda i: (0, 0)),
            pl.BlockSpec((1, DIM), lambda i: (0, 0)),
        ],
        out_specs=[
            pl.BlockSpec((br, DIM), lambda i: (i, 0)),
            pl.BlockSpec((br, DIM), lambda i: (i, 0)),
        ],
        out_shape=[
            jax.ShapeDtypeStruct((N, DIM), jnp.float32),
            jax.ShapeDtypeStruct((N, DIM), jnp.float32),
        ],
    )(emb1, emb2, W1, b1.reshape(1, DIM), W2, b2.reshape(1, DIM))


def _score_body(nv1_ref, nv2_ref, out_ref):
    dn = (((1,), (1,)), ((), ()))
    u = lax.dot_general(nv1_ref[...], nv2_ref[...], dn,
                        preferred_element_type=jnp.float32)
    out_ref[...] = 1.0 / (1.0 + jnp.exp(-A2 * u))


def _scores(nv1, nv2):
    br, bc = 256, 2048
    grid = (N // br + (N % br != 0), N // bc + (N % bc != 0))
    return pl.pallas_call(
        _score_body,
        grid=grid,
        in_specs=[
            pl.BlockSpec((br, DIM), lambda i, j: (i, 0)),
            pl.BlockSpec((bc, DIM), lambda i, j: (j, 0)),
        ],
        out_specs=pl.BlockSpec((br, bc), lambda i, j: (i, j)),
        out_shape=jax.ShapeDtypeStruct((N, N), jnp.float32),
    )(nv1, nv2)


def _hist_pass(shift, width, pmask):
    binmask = (1 << width) - 1
    if pmask >= 1 << 31:
        pmask -= 1 << 32
    mesh = plsc.VectorSubcoreMesh(core_axis_name="c", subcore_axis_name="s",
                                  num_cores=NC, num_subcores=NS)
    nextra = NWIN - (NWIN // NWORKERS) * NWORKERS
    nbase = NWIN // NWORKERS

    hwords = NCOPY * LANES * NBINS

    @functools.partial(
        pl.kernel,
        out_type=jax.ShapeDtypeStruct((NWORKERS * hwords,), jnp.int32),
        mesh=mesh,
        compiler_params=pltpu.CompilerParams(needs_layout_passes=False),
        scratch_types=[
            pltpu.VMEM((WIN,), jnp.float32),
            pltpu.VMEM((WIN,), jnp.float32),
            pltpu.VMEM((LANES,), jnp.int32),
            pltpu.VMEM((hwords,), jnp.int32),
            pltpu.SemaphoreType.DMA,
            pltpu.SemaphoreType.DMA,
        ],
    )
    def hist_kernel(s_hbm, pval_hbm, out_hbm, buf0, buf1, pvalv_ref, hist,
                    sem0, sem1):
        cid = lax.axis_index("c")
        sid = lax.axis_index("s")
        wid = sid * NC + cid

        def zero_body(i, carry):
            hist[pl.ds(i * LANES, LANES)] = jnp.zeros((LANES,), jnp.int32)
            return carry

        lax.fori_loop(0, hwords // LANES, zero_body, 0)

        pltpu.sync_copy(pval_hbm, pvalv_ref)
        pval = pvalv_ref[...]
        pm = jnp.full((LANES,), pmask, jnp.int32)
        shv = jnp.full((LANES,), shift, jnp.int32)
        bm = jnp.full((LANES,), binmask, jnp.int32)
        lane_off = lax.iota(jnp.int32, LANES) * NBINS
        ones = jnp.ones((LANES,), jnp.int32)
        nwin_w = nbase + jnp.where(wid < nextra, 1, 0)

        def src(g):
            win = wid + g * NWORKERS
            return s_hbm.at[pl.ds(win * WIN, WIN)]

        def start(g, buf, sem):
            pltpu.async_copy(src(g), buf, sem)

        def process(g, buf, sem):
            pltpu.make_async_copy(src(g), buf, sem).wait()

            def vec_body(jj, c2):
                for u in range(UNROLL):
                    j = jj * UNROLL + u
                    x = buf[pl.ds(j * LANES, LANES)]
                    key = plsc.bitcast(x, jnp.int32)
                    b = lax.shift_right_logical(key, shv) & bm
                    idx = b + lane_off + (u % NCOPY) * (LANES * NBINS)
                    if pmask == 0:
                        plsc.addupdate_scatter(hist, [idx], ones)
                    else:
                        sel = (key & pm) == pval
                        plsc.addupdate_scatter(hist, [idx], ones, mask=sel)
                return c2

            lax.fori_loop(0, WIN // LANES // UNROLL, vec_body, 0)

        start(0, buf0, sem0)

        def win_body(g, carry):
            even = (g % 2) == 0
            nxt = g + 1
            has_nxt = nxt < nwin_w

            @pl.when(jnp.logical_and(has_nxt, even))
            def _():
                start(nxt, buf1, sem1)

            @pl.when(jnp.logical_and(has_nxt, jnp.logical_not(even)))
            def _():
                start(nxt, buf0, sem0)

            @pl.when(even)
            def _():
                process(g, buf0, sem0)

            @pl.when(jnp.logical_not(even))
            def _():
                process(g, buf1, sem1)

            return carry

        lax.fori_loop(0, nwin_w, win_body, 0)
        pltpu.sync_copy(hist, out_hbm.at[pl.ds(wid * hwords, hwords)])

    return hist_kernel


_HIST_PASSES = [
    (21, 11, 0x00000000),
    (10, 11, 0xFFE00000),
    (0, 10, 0xFFFFFC00),
]
_HIST_KERNELS = None


def _get_hist_kernels():
    global _HIST_KERNELS
    if _HIST_KERNELS is None:
        _HIST_KERNELS = [_hist_pass(*p) for p in _HIST_PASSES]
    return _HIST_KERNELS


def _select_body(shift, hists_ref, carry_ref, out_ref):
    h = jnp.sum(hists_ref[...], axis=0, keepdims=True)
    iota = lax.broadcasted_iota(jnp.int32, (1, NBINS), 1)
    c = h
    sh = 1
    while sh < NBINS:
        r = pltpu.roll(c, sh, axis=1)
        c = c + jnp.where(iota >= sh, r, 0)
        sh *= 2
    tot = jnp.sum(h, axis=1, keepdims=True)
    cnt_gt = tot - c
    cnt_ge = cnt_gt + h
    krem = carry_ref[1]
    sel = (cnt_gt < krem) & (cnt_ge >= krem)
    bstar = jnp.max(jnp.where(sel, iota, -1), axis=(0, 1))
    cgt = jnp.max(jnp.where(sel, cnt_gt, -1), axis=(0, 1))
    out_ref[0] = carry_ref[0] + bstar * (1 << shift)
    out_ref[1] = krem - cgt
    out_ref[2] = 0
    out_ref[3] = 0


def _select_bin(shift, hists, carry):
    return pl.pallas_call(
        functools.partial(_select_body, shift),
        in_specs=[
            pl.BlockSpec((NWORKERS * NCOPY * LANES, NBINS), lambda: (0, 0)),
            pl.BlockSpec(memory_space=pltpu.SMEM),
        ],
        out_specs=pl.BlockSpec(memory_space=pltpu.SMEM),
        out_shape=jax.ShapeDtypeStruct((4,), jnp.int32),
    )(hists, carry)


def _mask_body(br, bc, s_ref, tk_ref, out_ref):
    i = pl.program_id(0)
    j = pl.program_id(1)
    s = s_ref[...]
    key = lax.bitcast_convert_type(s, jnp.int32)
    kept = jnp.where(key >= tk_ref[0], s, 0.0)
    row = i * br + lax.broadcasted_iota(jnp.int32, (br, bc), 0)
    col = j * bc + lax.broadcasted_iota(jnp.int32, (br, bc), 1)
    out_ref[...] = jnp.where(row == col, jnp.maximum(kept, 0.5), kept)


def _mask(S, tkey):
    br, bc = 256, 2048
    grid = (N // br + (N % br != 0), N // bc + (N % bc != 0))
    return pl.pallas_call(
        functools.partial(_mask_body, br, bc),
        grid=grid,
        in_specs=[
            pl.BlockSpec((br, bc), lambda i, j: (i, j)),
            pl.BlockSpec(memory_space=pltpu.SMEM),
        ],
        out_specs=pl.BlockSpec((br, bc), lambda i, j: (i, j)),
        out_shape=jax.ShapeDtypeStruct((N, N), jnp.float32),
    )(S, tkey)


def kernel(emb1, emb2, W1, b1, W2, b2):
    nv1, nv2 = _nodevecs(emb1, emb2, W1, b1, W2, b2)
    S = _scores(nv1, nv2)
    flat = S.reshape(TOTAL)
    carry = jnp.array([0, NUM_EDGES, 0, 0], jnp.int32)
    for (shift, _w, _m), hk in zip(_HIST_PASSES, _get_hist_kernels()):
        pvalv = jnp.broadcast_to(carry[0], (LANES,)).astype(jnp.int32)
        hists = hk(flat, pvalv)
        carry = _select_bin(
            shift, hists.reshape(NWORKERS * NCOPY * LANES, NBINS), carry)
    return _mask(S, carry)


# speedup vs baseline: 39.8819x; 1.1277x over previous
import functools

import jax
import jax.numpy as jnp
from jax import lax
from jax.experimental import pallas as pl
from jax.experimental.pallas import tpu as pltpu
from jax.experimental.pallas import tpu_sc as plsc

N = 10000
DIM = 128
NUM_EDGES = 320000
A1 = 0.1
A2 = 2.0

NC = 2
NS = 16
LANES = 16
NWORKERS = NC * NS

TOTAL = N * N
WIN = 16000
NWIN = TOTAL // WIN
NBINS = 1024
HSTRIDE = NBINS + 1
NCOPY = 4
UNROLL = 8


def _nodevec_body(emb1_ref, emb2_ref, w1_ref, b1_ref, w2_ref, b2_ref,
                  nv1_ref, nv2_ref):
    dn = (((1,), (1,)), ((), ()))
    z1 = lax.dot_general(emb1_ref[...], w1_ref[...], dn,
                         preferred_element_type=jnp.float32)
    nv1_ref[...] = jnp.tanh(A1 * (z1 + b1_ref[...]))
    z2 = lax.dot_general(emb2_ref[...], w2_ref[...], dn,
                         preferred_element_type=jnp.float32)
    nv2_ref[...] = jnp.tanh(A1 * (z2 + b2_ref[...]))


def _nodevecs(emb1, emb2, W1, b1, W2, b2):
    br = 1000
    grid = (N // br,)
    return pl.pallas_call(
        _nodevec_body,
        grid=grid,
        in_specs=[
            pl.BlockSpec((br, DIM), lambda i: (i, 0)),
            pl.BlockSpec((br, DIM), lambda i: (i, 0)),
            pl.BlockSpec((DIM, DIM), lambda i: (0, 0)),
            pl.BlockSpec((1, DIM), lambda i: (0, 0)),
            pl.BlockSpec((DIM, DIM), lambda i: (0, 0)),
            pl.BlockSpec((1, DIM), lambda i: (0, 0)),
        ],
        out_specs=[
            pl.BlockSpec((br, DIM), lambda i: (i, 0)),
            pl.BlockSpec((br, DIM), lambda i: (i, 0)),
        ],
        out_shape=[
            jax.ShapeDtypeStruct((N, DIM), jnp.float32),
            jax.ShapeDtypeStruct((N, DIM), jnp.float32),
        ],
    )(emb1, emb2, W1, b1.reshape(1, DIM), W2, b2.reshape(1, DIM))


def _score_body(nv1_ref, nv2_ref, out_ref):
    dn = (((1,), (1,)), ((), ()))
    u = lax.dot_general(nv1_ref[...], nv2_ref[...], dn,
                        preferred_element_type=jnp.float32)
    out_ref[...] = 1.0 / (1.0 + jnp.exp(-A2 * u))


def _scores(nv1, nv2):
    br, bc = 256, 2048
    grid = (N // br + (N % br != 0), N // bc + (N % bc != 0))
    return pl.pallas_call(
        _score_body,
        grid=grid,
        in_specs=[
            pl.BlockSpec((br, DIM), lambda i, j: (i, 0)),
            pl.BlockSpec((bc, DIM), lambda i, j: (j, 0)),
        ],
        out_specs=pl.BlockSpec((br, bc), lambda i, j: (i, j)),
        out_shape=jax.ShapeDtypeStruct((N, N), jnp.float32),
    )(nv1, nv2)


def _hist_pass(shift, width, pmask):
    binmask = (1 << width) - 1
    if pmask >= 1 << 31:
        pmask -= 1 << 32
    mesh = plsc.VectorSubcoreMesh(core_axis_name="c", subcore_axis_name="s",
                                  num_cores=NC, num_subcores=NS)
    nextra = NWIN - (NWIN // NWORKERS) * NWORKERS
    nbase = NWIN // NWORKERS

    hwords = NCOPY * LANES * HSTRIDE

    @functools.partial(
        pl.kernel,
        out_type=jax.ShapeDtypeStruct((NWORKERS * hwords,), jnp.int32),
        mesh=mesh,
        compiler_params=pltpu.CompilerParams(needs_layout_passes=False),
        scratch_types=[
            pltpu.VMEM((WIN,), jnp.float32),
            pltpu.VMEM((WIN,), jnp.float32),
            pltpu.VMEM((LANES,), jnp.int32),
            pltpu.VMEM((hwords,), jnp.int32),
            pltpu.SemaphoreType.DMA,
            pltpu.SemaphoreType.DMA,
        ],
    )
    def hist_kernel(s_hbm, pval_hbm, out_hbm, buf0, buf1, pvalv_ref, hist,
                    sem0, sem1):
        cid = lax.axis_index("c")
        sid = lax.axis_index("s")
        wid = sid * NC + cid

        def zero_body(i, carry):
            hist[pl.ds(i * LANES, LANES)] = jnp.zeros((LANES,), jnp.int32)
            return carry

        lax.fori_loop(0, hwords // LANES, zero_body, 0)

        pltpu.sync_copy(pval_hbm, pvalv_ref)
        pval = pvalv_ref[...]
        pm = jnp.full((LANES,), pmask, jnp.int32)
        shv = jnp.full((LANES,), shift, jnp.int32)
        bm = jnp.full((LANES,), binmask, jnp.int32)
        lane_off = lax.iota(jnp.int32, LANES) * HSTRIDE
        ones = jnp.ones((LANES,), jnp.int32)
        nwin_w = nbase + jnp.where(wid < nextra, 1, 0)

        def src(g):
            win = wid + g * NWORKERS
            return s_hbm.at[pl.ds(win * WIN, WIN)]

        def start(g, buf, sem):
            pltpu.async_copy(src(g), buf, sem)

        def process(g, buf, sem):
            pltpu.make_async_copy(src(g), buf, sem).wait()

            def vec_body(jj, c2):
                for u in range(UNROLL):
                    j = jj * UNROLL + u
                    x = buf[pl.ds(j * LANES, LANES)]
                    key = plsc.bitcast(x, jnp.int32)
                    b = lax.shift_right_logical(key, shv) & bm
                    idx = b + lane_off + (u % NCOPY) * (LANES * HSTRIDE)
                    if pmask == 0:
                        plsc.addupdate_scatter(hist, [idx], ones)
                    else:
                        sel = (key & pm) == pval
                        plsc.addupdate_scatter(hist, [idx], ones, mask=sel)
                return c2

            lax.fori_loop(0, WIN // LANES // UNROLL, vec_body, 0)

        start(0, buf0, sem0)

        def win_body(g, carry):
            even = (g % 2) == 0
            nxt = g + 1
            has_nxt = nxt < nwin_w

            @pl.when(jnp.logical_and(has_nxt, even))
            def _():
                start(nxt, buf1, sem1)

            @pl.when(jnp.logical_and(has_nxt, jnp.logical_not(even)))
            def _():
                start(nxt, buf0, sem0)

            @pl.when(even)
            def _():
                process(g, buf0, sem0)

            @pl.when(jnp.logical_not(even))
            def _():
                process(g, buf1, sem1)

            return carry

        lax.fori_loop(0, nwin_w, win_body, 0)
        pltpu.sync_copy(hist, out_hbm.at[pl.ds(wid * hwords, hwords)])

    return hist_kernel


_HIST_PASSES = [
    (20, 10, 0x00000000),
    (10, 10, 0xFFF00000),
    (0, 10, 0xFFFFFC00),
]
_HIST_KERNELS = None


def _get_hist_kernels():
    global _HIST_KERNELS
    if _HIST_KERNELS is None:
        _HIST_KERNELS = [_hist_pass(*p) for p in _HIST_PASSES]
    return _HIST_KERNELS


def _select_body(shift, hists_ref, carry_ref, out_ref):
    h = jnp.sum(hists_ref[...], axis=0, keepdims=True)
    iota = lax.broadcasted_iota(jnp.int32, (1, NBINS), 1)
    c = h
    sh = 1
    while sh < NBINS:
        r = pltpu.roll(c, sh, axis=1)
        c = c + jnp.where(iota >= sh, r, 0)
        sh *= 2
    tot = jnp.sum(h, axis=1, keepdims=True)
    cnt_gt = tot - c
    cnt_ge = cnt_gt + h
    krem = carry_ref[1]
    sel = (cnt_gt < krem) & (cnt_ge >= krem)
    bstar = jnp.max(jnp.where(sel, iota, -1), axis=(0, 1))
    cgt = jnp.max(jnp.where(sel, cnt_gt, -1), axis=(0, 1))
    out_ref[0] = carry_ref[0] + bstar * (1 << shift)
    out_ref[1] = krem - cgt
    out_ref[2] = 0
    out_ref[3] = 0


def _select_bin(shift, hists, carry):
    return pl.pallas_call(
        functools.partial(_select_body, shift),
        in_specs=[
            pl.BlockSpec((NWORKERS * NCOPY * LANES, NBINS), lambda: (0, 0)),
            pl.BlockSpec(memory_space=pltpu.SMEM),
        ],
        out_specs=pl.BlockSpec(memory_space=pltpu.SMEM),
        out_shape=jax.ShapeDtypeStruct((4,), jnp.int32),
    )(hists, carry)


def _mask_body(br, bc, s_ref, tk_ref, out_ref):
    i = pl.program_id(0)
    j = pl.program_id(1)
    s = s_ref[...]
    key = lax.bitcast_convert_type(s, jnp.int32)
    kept = jnp.where(key >= tk_ref[0], s, 0.0)
    row = i * br + lax.broadcasted_iota(jnp.int32, (br, bc), 0)
    col = j * bc + lax.broadcasted_iota(jnp.int32, (br, bc), 1)
    out_ref[...] = jnp.where(row == col, jnp.maximum(kept, 0.5), kept)


def _mask(S, tkey):
    br, bc = 256, 2048
    grid = (N // br + (N % br != 0), N // bc + (N % bc != 0))
    return pl.pallas_call(
        functools.partial(_mask_body, br, bc),
        grid=grid,
        in_specs=[
            pl.BlockSpec((br, bc), lambda i, j: (i, j)),
            pl.BlockSpec(memory_space=pltpu.SMEM),
        ],
        out_specs=pl.BlockSpec((br, bc), lambda i, j: (i, j)),
        out_shape=jax.ShapeDtypeStruct((N, N), jnp.float32),
    )(S, tkey)


def kernel(emb1, emb2, W1, b1, W2, b2):
    nv1, nv2 = _nodevecs(emb1, emb2, W1, b1, W2, b2)
    S = _scores(nv1, nv2)
    flat = S.reshape(TOTAL)
    carry = jnp.array([0, NUM_EDGES, 0, 0], jnp.int32)
    for (shift, _w, _m), hk in zip(_HIST_PASSES, _get_hist_kernels()):
        pvalv = jnp.broadcast_to(carry[0], (LANES,)).astype(jnp.int32)
        hists = hk(flat, pvalv)
        hists = hists.reshape(NWORKERS * NCOPY * LANES, HSTRIDE)[:, :NBINS]
        carry = _select_bin(shift, hists, carry)
    return _mask(S, carry)


# speedup vs baseline: 122.3607x; 3.0681x over previous
import functools

import jax
import jax.numpy as jnp
from jax import lax
from jax.experimental import pallas as pl
from jax.experimental.pallas import tpu as pltpu
from jax.experimental.pallas import tpu_sc as plsc

N = 10000
DIM = 128
NUM_EDGES = 320000
A1 = 0.1
A2 = 2.0

NC = 2
NS = 16
LANES = 16
NWORKERS = NC * NS

TOTAL = N * N
WIN = 16000
NWIN = TOTAL // WIN
NBINS = 1024
HSTRIDE = NBINS + 1
NCOPY = 4
UNROLL = 8


def _nodevec_body(emb1_ref, emb2_ref, w1_ref, b1_ref, w2_ref, b2_ref,
                  nv1_ref, nv2_ref):
    dn = (((1,), (1,)), ((), ()))
    z1 = lax.dot_general(emb1_ref[...], w1_ref[...], dn,
                         preferred_element_type=jnp.float32)
    nv1_ref[...] = jnp.tanh(A1 * (z1 + b1_ref[...]))
    z2 = lax.dot_general(emb2_ref[...], w2_ref[...], dn,
                         preferred_element_type=jnp.float32)
    nv2_ref[...] = jnp.tanh(A1 * (z2 + b2_ref[...]))


def _nodevecs(emb1, emb2, W1, b1, W2, b2):
    br = 1000
    grid = (N // br,)
    return pl.pallas_call(
        _nodevec_body,
        grid=grid,
        in_specs=[
            pl.BlockSpec((br, DIM), lambda i: (i, 0)),
            pl.BlockSpec((br, DIM), lambda i: (i, 0)),
            pl.BlockSpec((DIM, DIM), lambda i: (0, 0)),
            pl.BlockSpec((1, DIM), lambda i: (0, 0)),
            pl.BlockSpec((DIM, DIM), lambda i: (0, 0)),
            pl.BlockSpec((1, DIM), lambda i: (0, 0)),
        ],
        out_specs=[
            pl.BlockSpec((br, DIM), lambda i: (i, 0)),
            pl.BlockSpec((br, DIM), lambda i: (i, 0)),
        ],
        out_shape=[
            jax.ShapeDtypeStruct((N, DIM), jnp.float32),
            jax.ShapeDtypeStruct((N, DIM), jnp.float32),
        ],
    )(emb1, emb2, W1, b1.reshape(1, DIM), W2, b2.reshape(1, DIM))


def _score_body(nv1_ref, nv2_ref, out_ref):
    dn = (((1,), (1,)), ((), ()))
    u = lax.dot_general(nv1_ref[...], nv2_ref[...], dn,
                        preferred_element_type=jnp.float32)
    out_ref[...] = 1.0 / (1.0 + jnp.exp(-A2 * u))


def _scores(nv1, nv2):
    br, bc = 256, 2048
    grid = (N // br + (N % br != 0), N // bc + (N % bc != 0))
    return pl.pallas_call(
        _score_body,
        grid=grid,
        in_specs=[
            pl.BlockSpec((br, DIM), lambda i, j: (i, 0)),
            pl.BlockSpec((bc, DIM), lambda i, j: (j, 0)),
        ],
        out_specs=pl.BlockSpec((br, bc), lambda i, j: (i, j)),
        out_shape=jax.ShapeDtypeStruct((N, N), jnp.float32),
    )(nv1, nv2)


def _hist_pass(shift, width, pmask):
    binmask = (1 << width) - 1
    if pmask >= 1 << 31:
        pmask -= 1 << 32
    mesh = plsc.VectorSubcoreMesh(core_axis_name="c", subcore_axis_name="s",
                                  num_cores=NC, num_subcores=NS)
    nextra = NWIN - (NWIN // NWORKERS) * NWORKERS
    nbase = NWIN // NWORKERS

    hwords = NCOPY * LANES * HSTRIDE

    @functools.partial(
        pl.kernel,
        out_type=jax.ShapeDtypeStruct((NWORKERS * hwords,), jnp.int32),
        mesh=mesh,
        compiler_params=pltpu.CompilerParams(needs_layout_passes=False),
        scratch_types=[
            pltpu.VMEM((WIN,), jnp.float32),
            pltpu.VMEM((WIN,), jnp.float32),
            pltpu.VMEM((LANES,), jnp.int32),
            pltpu.VMEM((hwords,), jnp.int32),
            pltpu.SemaphoreType.DMA,
            pltpu.SemaphoreType.DMA,
        ],
    )
    def hist_kernel(s_hbm, pval_hbm, out_hbm, buf0, buf1, pvalv_ref, hist,
                    sem0, sem1):
        cid = lax.axis_index("c")
        sid = lax.axis_index("s")
        wid = sid * NC + cid

        def zero_body(i, carry):
            hist[pl.ds(i * LANES, LANES)] = jnp.zeros((LANES,), jnp.int32)
            return carry

        lax.fori_loop(0, hwords // LANES, zero_body, 0)

        pltpu.sync_copy(pval_hbm, pvalv_ref)
        pval = pvalv_ref[...]
        pm = jnp.full((LANES,), pmask, jnp.int32)
        shv = jnp.full((LANES,), shift, jnp.int32)
        bm = jnp.full((LANES,), binmask, jnp.int32)
        lane_off = lax.iota(jnp.int32, LANES) * HSTRIDE
        ones = jnp.ones((LANES,), jnp.int32)
        nwin_w = nbase + jnp.where(wid < nextra, 1, 0)

        def src(g):
            win = wid + g * NWORKERS
            return s_hbm.at[pl.ds(win * WIN, WIN)]

        def start(g, buf, sem):
            pltpu.async_copy(src(g), buf, sem)

        def process(g, buf, sem):
            pltpu.make_async_copy(src(g), buf, sem).wait()

            @plsc.parallel_loop(0, WIN // LANES, step=1, unroll=UNROLL)
            def _vec(j):
                x = buf[pl.ds(j * LANES, LANES)]
                key = plsc.bitcast(x, jnp.int32)
                b = lax.shift_right_logical(key, shv) & bm
                idx = b + lane_off + (j % NCOPY) * (LANES * HSTRIDE)
                if pmask == 0:
                    plsc.addupdate_scatter(hist, [idx], ones)
                else:
                    sel = (key & pm) == pval
                    plsc.addupdate_scatter(hist, [idx], ones, mask=sel)

        start(0, buf0, sem0)

        def win_body(g, carry):
            even = (g % 2) == 0
            nxt = g + 1
            has_nxt = nxt < nwin_w

            @pl.when(jnp.logical_and(has_nxt, even))
            def _():
                start(nxt, buf1, sem1)

            @pl.when(jnp.logical_and(has_nxt, jnp.logical_not(even)))
            def _():
                start(nxt, buf0, sem0)

            @pl.when(even)
            def _():
                process(g, buf0, sem0)

            @pl.when(jnp.logical_not(even))
            def _():
                process(g, buf1, sem1)

            return carry

        lax.fori_loop(0, nwin_w, win_body, 0)
        pltpu.sync_copy(hist, out_hbm.at[pl.ds(wid * hwords, hwords)])

    return hist_kernel


_HIST_PASSES = [
    (20, 10, 0x00000000),
    (10, 10, 0xFFF00000),
    (0, 10, 0xFFFFFC00),
]
_HIST_KERNELS = None


def _get_hist_kernels():
    global _HIST_KERNELS
    if _HIST_KERNELS is None:
        _HIST_KERNELS = [_hist_pass(*p) for p in _HIST_PASSES]
    return _HIST_KERNELS


def _select_body(shift, hists_ref, carry_ref, out_ref):
    h = jnp.sum(hists_ref[...], axis=0, keepdims=True)
    iota = lax.broadcasted_iota(jnp.int32, (1, NBINS), 1)
    c = h
    sh = 1
    while sh < NBINS:
        r = pltpu.roll(c, sh, axis=1)
        c = c + jnp.where(iota >= sh, r, 0)
        sh *= 2
    tot = jnp.sum(h, axis=1, keepdims=True)
    cnt_gt = tot - c
    cnt_ge = cnt_gt + h
    krem = carry_ref[1]
    sel = (cnt_gt < krem) & (cnt_ge >= krem)
    bstar = jnp.max(jnp.where(sel, iota, -1), axis=(0, 1))
    cgt = jnp.max(jnp.where(sel, cnt_gt, -1), axis=(0, 1))
    out_ref[0] = carry_ref[0] + bstar * (1 << shift)
    out_ref[1] = krem - cgt
    out_ref[2] = 0
    out_ref[3] = 0


def _select_bin(shift, hists, carry):
    return pl.pallas_call(
        functools.partial(_select_body, shift),
        in_specs=[
            pl.BlockSpec((NWORKERS * NCOPY * LANES, NBINS), lambda: (0, 0)),
            pl.BlockSpec(memory_space=pltpu.SMEM),
        ],
        out_specs=pl.BlockSpec(memory_space=pltpu.SMEM),
        out_shape=jax.ShapeDtypeStruct((4,), jnp.int32),
    )(hists, carry)


def _mask_body(br, bc, s_ref, tk_ref, out_ref):
    i = pl.program_id(0)
    j = pl.program_id(1)
    s = s_ref[...]
    key = lax.bitcast_convert_type(s, jnp.int32)
    kept = jnp.where(key >= tk_ref[0], s, 0.0)
    row = i * br + lax.broadcasted_iota(jnp.int32, (br, bc), 0)
    col = j * bc + lax.broadcasted_iota(jnp.int32, (br, bc), 1)
    out_ref[...] = jnp.where(row == col, jnp.maximum(kept, 0.5), kept)


def _mask(S, tkey):
    br, bc = 256, 2048
    grid = (N // br + (N % br != 0), N // bc + (N % bc != 0))
    return pl.pallas_call(
        functools.partial(_mask_body, br, bc),
        grid=grid,
        in_specs=[
            pl.BlockSpec((br, bc), lambda i, j: (i, j)),
            pl.BlockSpec(memory_space=pltpu.SMEM),
        ],
        out_specs=pl.BlockSpec((br, bc), lambda i, j: (i, j)),
        out_shape=jax.ShapeDtypeStruct((N, N), jnp.float32),
    )(S, tkey)


def kernel(emb1, emb2, W1, b1, W2, b2):
    nv1, nv2 = _nodevecs(emb1, emb2, W1, b1, W2, b2)
    S = _scores(nv1, nv2)
    flat = S.reshape(TOTAL)
    carry = jnp.array([0, NUM_EDGES, 0, 0], jnp.int32)
    for (shift, _w, _m), hk in zip(_HIST_PASSES, _get_hist_kernels()):
        pvalv = jnp.broadcast_to(carry[0], (LANES,)).astype(jnp.int32)
        hists = hk(flat, pvalv)
        hists = hists.reshape(NWORKERS * NCOPY * LANES, HSTRIDE)[:, :NBINS]
        carry = _select_bin(shift, hists, carry)
    return _mask(S, carry)


# speedup vs baseline: 151.3567x; 1.2370x over previous
import functools

import jax
import jax.numpy as jnp
from jax import lax
from jax.experimental import pallas as pl
from jax.experimental.pallas import tpu as pltpu
from jax.experimental.pallas import tpu_sc as plsc

N = 10000
DIM = 128
NUM_EDGES = 320000
A1 = 0.1
A2 = 2.0

NC = 2
NS = 16
LANES = 16
NWORKERS = NC * NS

TOTAL = N * N
NPAD = 10112
NBINS = 1024
HSTRIDE = NBINS + 1
NCOPY = 4
UNROLL = 8


def _nodevec_body(emb1_ref, emb2_ref, w1_ref, b1_ref, w2_ref, b2_ref,
                  nv1_ref, nv2_ref):
    dn = (((1,), (1,)), ((), ()))
    z1 = lax.dot_general(emb1_ref[...], w1_ref[...], dn,
                         preferred_element_type=jnp.float32)
    nv1_ref[...] = jnp.tanh(A1 * (z1 + b1_ref[...]))
    z2 = lax.dot_general(emb2_ref[...], w2_ref[...], dn,
                         preferred_element_type=jnp.float32)
    nv2_ref[...] = jnp.tanh(A1 * (z2 + b2_ref[...]))


def _nodevecs(emb1, emb2, W1, b1, W2, b2):
    br = 1000
    grid = (N // br,)
    return pl.pallas_call(
        _nodevec_body,
        grid=grid,
        in_specs=[
            pl.BlockSpec((br, DIM), lambda i: (i, 0)),
            pl.BlockSpec((br, DIM), lambda i: (i, 0)),
            pl.BlockSpec((DIM, DIM), lambda i: (0, 0)),
            pl.BlockSpec((1, DIM), lambda i: (0, 0)),
            pl.BlockSpec((DIM, DIM), lambda i: (0, 0)),
            pl.BlockSpec((1, DIM), lambda i: (0, 0)),
        ],
        out_specs=[
            pl.BlockSpec((br, DIM), lambda i: (i, 0)),
            pl.BlockSpec((br, DIM), lambda i: (i, 0)),
        ],
        out_shape=[
            jax.ShapeDtypeStruct((N, DIM), jnp.float32),
            jax.ShapeDtypeStruct((N, DIM), jnp.float32),
        ],
    )(emb1, emb2, W1, b1.reshape(1, DIM), W2, b2.reshape(1, DIM))


def _score_body(br, nv1_ref, nv2_ref, out_ref):
    dn = (((1,), (1,)), ((), ()))
    u = lax.dot_general(nv1_ref[...], nv2_ref[...], dn,
                        preferred_element_type=jnp.float32)
    out_ref[:, pl.ds(0, N)] = 1.0 / (1.0 + jnp.exp(-A2 * u))
    out_ref[:, pl.ds(N, NPAD - N)] = jnp.zeros((br, NPAD - N), jnp.float32)


def _scores(nv1, nv2):
    br = 200
    grid = (N // br,)
    return pl.pallas_call(
        functools.partial(_score_body, br),
        grid=grid,
        in_specs=[
            pl.BlockSpec((br, DIM), lambda i: (i, 0)),
            pl.BlockSpec((N, DIM), lambda i: (0, 0)),
        ],
        out_specs=pl.BlockSpec((br, NPAD), lambda i: (i, 0)),
        out_shape=jax.ShapeDtypeStruct((N, NPAD), jnp.float32),
    )(nv1, nv2)


def _hist_pass(shift, width, pmask):
    binmask = (1 << width) - 1
    if pmask >= 1 << 31:
        pmask -= 1 << 32
    mesh = plsc.VectorSubcoreMesh(core_axis_name="c", subcore_axis_name="s",
                                  num_cores=NC, num_subcores=NS)
    nextra = N - (N // NWORKERS) * NWORKERS
    nbase = N // NWORKERS

    hwords = NCOPY * LANES * HSTRIDE

    @functools.partial(
        pl.kernel,
        out_type=jax.ShapeDtypeStruct((NWORKERS * hwords,), jnp.int32),
        mesh=mesh,
        compiler_params=pltpu.CompilerParams(needs_layout_passes=False),
        scratch_types=[
            pltpu.VMEM((NPAD,), jnp.float32),
            pltpu.VMEM((NPAD,), jnp.float32),
            pltpu.VMEM((LANES,), jnp.int32),
            pltpu.VMEM((hwords,), jnp.int32),
            pltpu.SemaphoreType.DMA,
            pltpu.SemaphoreType.DMA,
        ],
    )
    def hist_kernel(s_hbm, pval_hbm, out_hbm, buf0, buf1, pvalv_ref, hist,
                    sem0, sem1):
        cid = lax.axis_index("c")
        sid = lax.axis_index("s")
        wid = sid * NC + cid

        def zero_body(i, carry):
            hist[pl.ds(i * LANES, LANES)] = jnp.zeros((LANES,), jnp.int32)
            return carry

        lax.fori_loop(0, hwords // LANES, zero_body, 0)

        pltpu.sync_copy(pval_hbm, pvalv_ref)
        pval = pvalv_ref[...]
        pm = jnp.full((LANES,), pmask, jnp.int32)
        shv = jnp.full((LANES,), shift, jnp.int32)
        bm = jnp.full((LANES,), binmask, jnp.int32)
        lane_off = lax.iota(jnp.int32, LANES) * HSTRIDE
        ones = jnp.ones((LANES,), jnp.int32)
        nwin_w = nbase + jnp.where(wid < nextra, 1, 0)

        def src(g):
            return s_hbm.at[wid + g * NWORKERS]

        def start(g, buf, sem):
            pltpu.async_copy(src(g), buf, sem)

        def process(g, buf, sem):
            pltpu.make_async_copy(src(g), buf, sem).wait()

            @plsc.parallel_loop(0, NPAD // LANES, step=1, unroll=UNROLL)
            def _vec(j):
                x = buf[pl.ds(j * LANES, LANES)]
                key = plsc.bitcast(x, jnp.int32)
                b = lax.shift_right_logical(key, shv) & bm
                idx = b + lane_off + (j % NCOPY) * (LANES * HSTRIDE)
                if pmask == 0:
                    plsc.addupdate_scatter(hist, [idx], ones)
                else:
                    sel = (key & pm) == pval
                    plsc.addupdate_scatter(hist, [idx], ones, mask=sel)

        start(0, buf0, sem0)

        def win_body(g, carry):
            even = (g % 2) == 0
            nxt = g + 1
            has_nxt = nxt < nwin_w

            @pl.when(jnp.logical_and(has_nxt, even))
            def _():
                start(nxt, buf1, sem1)

            @pl.when(jnp.logical_and(has_nxt, jnp.logical_not(even)))
            def _():
                start(nxt, buf0, sem0)

            @pl.when(even)
            def _():
                process(g, buf0, sem0)

            @pl.when(jnp.logical_not(even))
            def _():
                process(g, buf1, sem1)

            return carry

        lax.fori_loop(0, nwin_w, win_body, 0)
        pltpu.sync_copy(hist, out_hbm.at[pl.ds(wid * hwords, hwords)])

    return hist_kernel


_HIST_PASSES = [
    (20, 10, 0x00000000),
    (10, 10, 0xFFF00000),
    (0, 10, 0xFFFFFC00),
]
_HIST_KERNELS = None


def _get_hist_kernels():
    global _HIST_KERNELS
    if _HIST_KERNELS is None:
        _HIST_KERNELS = [_hist_pass(*p) for p in _HIST_PASSES]
    return _HIST_KERNELS


def _select_body(shift, hists_ref, carry_ref, out_ref):
    h = jnp.sum(hists_ref[...], axis=0, keepdims=True)
    iota = lax.broadcasted_iota(jnp.int32, (1, NBINS), 1)
    c = h
    sh = 1
    while sh < NBINS:
        r = pltpu.roll(c, sh, axis=1)
        c = c + jnp.where(iota >= sh, r, 0)
        sh *= 2
    tot = jnp.sum(h, axis=1, keepdims=True)
    cnt_gt = tot - c
    cnt_ge = cnt_gt + h
    krem = carry_ref[1]
    sel = (cnt_gt < krem) & (cnt_ge >= krem)
    bstar = jnp.max(jnp.where(sel, iota, -1), axis=(0, 1))
    cgt = jnp.max(jnp.where(sel, cnt_gt, -1), axis=(0, 1))
    out_ref[0] = carry_ref[0] + bstar * (1 << shift)
    out_ref[1] = krem - cgt
    out_ref[2] = 0
    out_ref[3] = 0


def _select_bin(shift, hists, carry):
    return pl.pallas_call(
        functools.partial(_select_body, shift),
        in_specs=[
            pl.BlockSpec((NWORKERS * NCOPY * LANES, NBINS), lambda: (0, 0)),
            pl.BlockSpec(memory_space=pltpu.SMEM),
        ],
        out_specs=pl.BlockSpec(memory_space=pltpu.SMEM),
        out_shape=jax.ShapeDtypeStruct((4,), jnp.int32),
    )(hists, carry)


def _mask_body(br, bc, s_ref, tk_ref, out_ref):
    i = pl.program_id(0)
    j = pl.program_id(1)
    s = s_ref[...]
    key = lax.bitcast_convert_type(s, jnp.int32)
    kept = jnp.where(key >= tk_ref[0], s, 0.0)
    row = i * br + lax.broadcasted_iota(jnp.int32, (br, bc), 0)
    col = j * bc + lax.broadcasted_iota(jnp.int32, (br, bc), 1)
    out_ref[...] = jnp.where(row == col, jnp.maximum(kept, 0.5), kept)


def _mask(S, tkey):
    br, bc = 256, 2048
    grid = (N // br + (N % br != 0), N // bc + (N % bc != 0))
    return pl.pallas_call(
        functools.partial(_mask_body, br, bc),
        grid=grid,
        in_specs=[
            pl.BlockSpec((br, bc), lambda i, j: (i, j)),
            pl.BlockSpec(memory_space=pltpu.SMEM),
        ],
        out_specs=pl.BlockSpec((br, bc), lambda i, j: (i, j)),
        out_shape=jax.ShapeDtypeStruct((N, N), jnp.float32),
    )(S, tkey)


def kernel(emb1, emb2, W1, b1, W2, b2):
    nv1, nv2 = _nodevecs(emb1, emb2, W1, b1, W2, b2)
    S = _scores(nv1, nv2)
    carry = jnp.array([0, NUM_EDGES, 0, 0], jnp.int32)
    for (shift, _w, _m), hk in zip(_HIST_PASSES, _get_hist_kernels()):
        pvalv = jnp.broadcast_to(carry[0], (LANES,)).astype(jnp.int32)
        hists = hk(S, pvalv)
        hists = hists.reshape(NWORKERS * NCOPY * LANES, HSTRIDE)[:, :NBINS]
        carry = _select_bin(shift, hists, carry)
    return _mask(S, carry)


# speedup vs baseline: 168.7551x; 1.1150x over previous
import functools

import jax
import jax.numpy as jnp
from jax import lax
from jax.experimental import pallas as pl
from jax.experimental.pallas import tpu as pltpu
from jax.experimental.pallas import tpu_sc as plsc

N = 10000
DIM = 128
NUM_EDGES = 320000
A1 = 0.1
A2 = 2.0

NC = 2
NS = 16
LANES = 16
NWORKERS = NC * NS

TOTAL = N * N
NPAD = 10112
NBINS = 1024
HSTRIDE = NBINS + 1
NCOPY = 4
UNROLL = 8


def _nodevec_body(emb1_ref, emb2_ref, w1_ref, b1_ref, w2_ref, b2_ref,
                  nv1_ref, nv2_ref):
    dn = (((1,), (1,)), ((), ()))
    z1 = lax.dot_general(emb1_ref[...], w1_ref[...], dn,
                         preferred_element_type=jnp.float32)
    nv1_ref[...] = jnp.tanh(A1 * (z1 + b1_ref[...]))
    z2 = lax.dot_general(emb2_ref[...], w2_ref[...], dn,
                         preferred_element_type=jnp.float32)
    nv2_ref[...] = jnp.tanh(A1 * (z2 + b2_ref[...]))


def _nodevecs(emb1, emb2, W1, b1, W2, b2):
    br = 1000
    grid = (N // br,)
    return pl.pallas_call(
        _nodevec_body,
        grid=grid,
        in_specs=[
            pl.BlockSpec((br, DIM), lambda i: (i, 0)),
            pl.BlockSpec((br, DIM), lambda i: (i, 0)),
            pl.BlockSpec((DIM, DIM), lambda i: (0, 0)),
            pl.BlockSpec((1, DIM), lambda i: (0, 0)),
            pl.BlockSpec((DIM, DIM), lambda i: (0, 0)),
            pl.BlockSpec((1, DIM), lambda i: (0, 0)),
        ],
        out_specs=[
            pl.BlockSpec((br, DIM), lambda i: (i, 0)),
            pl.BlockSpec((br, DIM), lambda i: (i, 0)),
        ],
        out_shape=[
            jax.ShapeDtypeStruct((N, DIM), jnp.float32),
            jax.ShapeDtypeStruct((N, DIM), jnp.float32),
        ],
    )(emb1, emb2, W1, b1.reshape(1, DIM), W2, b2.reshape(1, DIM))


def _score_body(br, nv1_ref, nv2_ref, out_ref):
    dn = (((1,), (1,)), ((), ()))
    u = lax.dot_general(nv1_ref[...], nv2_ref[...], dn,
                        preferred_element_type=jnp.float32)
    out_ref[:, pl.ds(0, N)] = 1.0 / (1.0 + jnp.exp(-A2 * u))
    out_ref[:, pl.ds(N, NPAD - N)] = jnp.zeros((br, NPAD - N), jnp.float32)


def _scores(nv1, nv2):
    br = 200
    grid = (N // br,)
    return pl.pallas_call(
        functools.partial(_score_body, br),
        grid=grid,
        in_specs=[
            pl.BlockSpec((br, DIM), lambda i: (i, 0)),
            pl.BlockSpec((N, DIM), lambda i: (0, 0)),
        ],
        out_specs=pl.BlockSpec((br, NPAD), lambda i: (i, 0)),
        out_shape=jax.ShapeDtypeStruct((N, NPAD), jnp.float32),
    )(nv1, nv2)


def _hist_pass(shift, width, pmask):
    binmask = (1 << width) - 1
    if pmask >= 1 << 31:
        pmask -= 1 << 32
    mesh = plsc.VectorSubcoreMesh(core_axis_name="c", subcore_axis_name="s",
                                  num_cores=NC, num_subcores=NS)
    nband = N // 2
    nextra = nband - (nband // NWORKERS) * NWORKERS
    nbase = nband // NWORKERS

    hwords = NCOPY * LANES * HSTRIDE

    @functools.partial(
        pl.kernel,
        out_type=jax.ShapeDtypeStruct((NWORKERS * hwords,), jnp.int32),
        mesh=mesh,
        compiler_params=pltpu.CompilerParams(needs_layout_passes=False),
        scratch_types=[
            pltpu.VMEM((2, NPAD), jnp.float32),
            pltpu.VMEM((2, NPAD), jnp.float32),
            pltpu.VMEM((LANES,), jnp.int32),
            pltpu.VMEM((hwords,), jnp.int32),
            pltpu.SemaphoreType.DMA,
            pltpu.SemaphoreType.DMA,
        ],
    )
    def hist_kernel(s_hbm, pval_hbm, out_hbm, buf0, buf1, pvalv_ref, hist,
                    sem0, sem1):
        cid = lax.axis_index("c")
        sid = lax.axis_index("s")
        wid = sid * NC + cid

        def zero_body(i, carry):
            hist[pl.ds(i * LANES, LANES)] = jnp.zeros((LANES,), jnp.int32)
            return carry

        lax.fori_loop(0, hwords // LANES, zero_body, 0)

        pltpu.sync_copy(pval_hbm, pvalv_ref)
        pval = pvalv_ref[...]
        pm = jnp.full((LANES,), pmask, jnp.int32)
        shv = jnp.full((LANES,), shift, jnp.int32)
        bm = jnp.full((LANES,), binmask, jnp.int32)
        lane_off = lax.iota(jnp.int32, LANES) * HSTRIDE
        ones = jnp.ones((LANES,), jnp.int32)
        nwin_w = nbase + jnp.where(wid < nextra, 1, 0)

        def src(g):
            return s_hbm.at[pl.ds((wid + g * NWORKERS) * 2, 2), :]

        def start(g, buf, sem):
            pltpu.async_copy(src(g), buf, sem)

        def process(g, buf, sem):
            pltpu.make_async_copy(src(g), buf, sem).wait()

            for r in range(2):
                @plsc.parallel_loop(0, NPAD // LANES, step=1, unroll=UNROLL)
                def _vec(j, r=r):
                    x = buf[r, pl.ds(j * LANES, LANES)]
                    key = plsc.bitcast(x, jnp.int32)
                    b = lax.shift_right_logical(key, shv) & bm
                    idx = b + lane_off + (j % NCOPY) * (LANES * HSTRIDE)
                    if pmask == 0:
                        plsc.addupdate_scatter(hist, [idx], ones)
                    else:
                        sel = (key & pm) == pval
                        plsc.addupdate_scatter(hist, [idx], ones, mask=sel)

        start(0, buf0, sem0)

        def win_body(g, carry):
            even = (g % 2) == 0
            nxt = g + 1
            has_nxt = nxt < nwin_w

            @pl.when(jnp.logical_and(has_nxt, even))
            def _():
                start(nxt, buf1, sem1)

            @pl.when(jnp.logical_and(has_nxt, jnp.logical_not(even)))
            def _():
                start(nxt, buf0, sem0)

            @pl.when(even)
            def _():
                process(g, buf0, sem0)

            @pl.when(jnp.logical_not(even))
            def _():
                process(g, buf1, sem1)

            return carry

        lax.fori_loop(0, nwin_w, win_body, 0)
        pltpu.sync_copy(hist, out_hbm.at[pl.ds(wid * hwords, hwords)])

    return hist_kernel


_HIST_PASSES = [
    (20, 10, 0x00000000),
    (10, 10, 0xFFF00000),
    (0, 10, 0xFFFFFC00),
]
_HIST_KERNELS = None


def _get_hist_kernels():
    global _HIST_KERNELS
    if _HIST_KERNELS is None:
        _HIST_KERNELS = [_hist_pass(*p) for p in _HIST_PASSES]
    return _HIST_KERNELS


def _select_body(shift, hists_ref, carry_ref, out_ref):
    h = jnp.sum(hists_ref[...], axis=0, keepdims=True)
    iota = lax.broadcasted_iota(jnp.int32, (1, NBINS), 1)
    c = h
    sh = 1
    while sh < NBINS:
        r = pltpu.roll(c, sh, axis=1)
        c = c + jnp.where(iota >= sh, r, 0)
        sh *= 2
    tot = jnp.sum(h, axis=1, keepdims=True)
    cnt_gt = tot - c
    cnt_ge = cnt_gt + h
    krem = carry_ref[1]
    sel = (cnt_gt < krem) & (cnt_ge >= krem)
    bstar = jnp.max(jnp.where(sel, iota, -1), axis=(0, 1))
    cgt = jnp.max(jnp.where(sel, cnt_gt, -1), axis=(0, 1))
    out_ref[0] = carry_ref[0] + bstar * (1 << shift)
    out_ref[1] = krem - cgt
    out_ref[2] = 0
    out_ref[3] = 0


def _select_bin(shift, hists, carry):
    return pl.pallas_call(
        functools.partial(_select_body, shift),
        in_specs=[
            pl.BlockSpec((NWORKERS * NCOPY * LANES, NBINS), lambda: (0, 0)),
            pl.BlockSpec(memory_space=pltpu.SMEM),
        ],
        out_specs=pl.BlockSpec(memory_space=pltpu.SMEM),
        out_shape=jax.ShapeDtypeStruct((4,), jnp.int32),
    )(hists, carry)


def _mask_body(br, bc, s_ref, tk_ref, out_ref):
    i = pl.program_id(0)
    j = pl.program_id(1)
    s = s_ref[...]
    key = lax.bitcast_convert_type(s, jnp.int32)
    kept = jnp.where(key >= tk_ref[0], s, 0.0)
    row = i * br + lax.broadcasted_iota(jnp.int32, (br, bc), 0)
    col = j * bc + lax.broadcasted_iota(jnp.int32, (br, bc), 1)
    out_ref[...] = jnp.where(row == col, jnp.maximum(kept, 0.5), kept)


def _mask(S, tkey):
    br, bc = 256, 2048
    grid = (N // br + (N % br != 0), N // bc + (N % bc != 0))
    return pl.pallas_call(
        functools.partial(_mask_body, br, bc),
        grid=grid,
        in_specs=[
            pl.BlockSpec((br, bc), lambda i, j: (i, j)),
            pl.BlockSpec(memory_space=pltpu.SMEM),
        ],
        out_specs=pl.BlockSpec((br, bc), lambda i, j: (i, j)),
        out_shape=jax.ShapeDtypeStruct((N, N), jnp.float32),
    )(S, tkey)


def kernel(emb1, emb2, W1, b1, W2, b2):
    nv1, nv2 = _nodevecs(emb1, emb2, W1, b1, W2, b2)
    S = _scores(nv1, nv2)
    carry = jnp.array([0, NUM_EDGES, 0, 0], jnp.int32)
    for (shift, _w, _m), hk in zip(_HIST_PASSES, _get_hist_kernels()):
        pvalv = jnp.broadcast_to(carry[0], (LANES,)).astype(jnp.int32)
        hists = hk(S, pvalv)
        hists = hists.reshape(NWORKERS * NCOPY * LANES, HSTRIDE)[:, :NBINS]
        carry = _select_bin(shift, hists, carry)
    return _mask(S, carry)


# speedup vs baseline: 180.3203x; 1.0685x over previous
import functools

import jax
import jax.numpy as jnp
from jax import lax
from jax.experimental import pallas as pl
from jax.experimental.pallas import tpu as pltpu
from jax.experimental.pallas import tpu_sc as plsc

N = 10000
DIM = 128
NUM_EDGES = 320000
A1 = 0.1
A2 = 2.0

NC = 2
NS = 16
LANES = 16
NWORKERS = NC * NS

TOTAL = N * N
NPAD = 10112
NBINS = 1024
HSTRIDE = NBINS + 1
NCOPY = 2
UNROLL = 8
ROWS_PER_BAND = 4


def _nodevec_body(emb1_ref, emb2_ref, w1_ref, b1_ref, w2_ref, b2_ref,
                  nv1_ref, nv2_ref):
    dn = (((1,), (1,)), ((), ()))
    z1 = lax.dot_general(emb1_ref[...], w1_ref[...], dn,
                         preferred_element_type=jnp.float32)
    nv1_ref[...] = jnp.tanh(A1 * (z1 + b1_ref[...]))
    z2 = lax.dot_general(emb2_ref[...], w2_ref[...], dn,
                         preferred_element_type=jnp.float32)
    nv2_ref[...] = jnp.tanh(A1 * (z2 + b2_ref[...]))


def _nodevecs(emb1, emb2, W1, b1, W2, b2):
    br = 1000
    grid = (N // br,)
    return pl.pallas_call(
        _nodevec_body,
        grid=grid,
        in_specs=[
            pl.BlockSpec((br, DIM), lambda i: (i, 0)),
            pl.BlockSpec((br, DIM), lambda i: (i, 0)),
            pl.BlockSpec((DIM, DIM), lambda i: (0, 0)),
            pl.BlockSpec((1, DIM), lambda i: (0, 0)),
            pl.BlockSpec((DIM, DIM), lambda i: (0, 0)),
            pl.BlockSpec((1, DIM), lambda i: (0, 0)),
        ],
        out_specs=[
            pl.BlockSpec((br, DIM), lambda i: (i, 0)),
            pl.BlockSpec((br, DIM), lambda i: (i, 0)),
        ],
        out_shape=[
            jax.ShapeDtypeStruct((N, DIM), jnp.float32),
            jax.ShapeDtypeStruct((N, DIM), jnp.float32),
        ],
    )(emb1, emb2, W1, b1.reshape(1, DIM), W2, b2.reshape(1, DIM))


def _score_body(br, nv1_ref, nv2_ref, out_ref):
    dn = (((1,), (1,)), ((), ()))
    u = lax.dot_general(nv1_ref[...], nv2_ref[...], dn,
                        preferred_element_type=jnp.float32)
    out_ref[:, pl.ds(0, N)] = 1.0 / (1.0 + jnp.exp(-A2 * u))
    out_ref[:, pl.ds(N, NPAD - N)] = jnp.zeros((br, NPAD - N), jnp.float32)


def _scores(nv1, nv2):
    br = 200
    grid = (N // br,)
    return pl.pallas_call(
        functools.partial(_score_body, br),
        grid=grid,
        in_specs=[
            pl.BlockSpec((br, DIM), lambda i: (i, 0)),
            pl.BlockSpec((N, DIM), lambda i: (0, 0)),
        ],
        out_specs=pl.BlockSpec((br, NPAD), lambda i: (i, 0)),
        out_shape=jax.ShapeDtypeStruct((N, NPAD), jnp.float32),
    )(nv1, nv2)


def _hist_pass(shift, width, pmask):
    binmask = (1 << width) - 1
    if pmask >= 1 << 31:
        pmask -= 1 << 32
    mesh = plsc.VectorSubcoreMesh(core_axis_name="c", subcore_axis_name="s",
                                  num_cores=NC, num_subcores=NS)
    nband = N // ROWS_PER_BAND
    nextra = nband - (nband // NWORKERS) * NWORKERS
    nbase = nband // NWORKERS

    hwords = NCOPY * LANES * HSTRIDE

    @functools.partial(
        pl.kernel,
        out_type=jax.ShapeDtypeStruct((NWORKERS * hwords,), jnp.int32),
        mesh=mesh,
        compiler_params=pltpu.CompilerParams(needs_layout_passes=False),
        scratch_types=[
            pltpu.VMEM((ROWS_PER_BAND, NPAD), jnp.float32),
            pltpu.VMEM((ROWS_PER_BAND, NPAD), jnp.float32),
            pltpu.VMEM((LANES,), jnp.int32),
            pltpu.VMEM((hwords,), jnp.int32),
            pltpu.SemaphoreType.DMA,
            pltpu.SemaphoreType.DMA,
        ],
    )
    def hist_kernel(s_hbm, pval_hbm, out_hbm, buf0, buf1, pvalv_ref, hist,
                    sem0, sem1):
        cid = lax.axis_index("c")
        sid = lax.axis_index("s")
        wid = sid * NC + cid

        def zero_body(i, carry):
            hist[pl.ds(i * LANES, LANES)] = jnp.zeros((LANES,), jnp.int32)
            return carry

        lax.fori_loop(0, hwords // LANES, zero_body, 0)

        pltpu.sync_copy(pval_hbm, pvalv_ref)
        pval = pvalv_ref[...]
        pm = jnp.full((LANES,), pmask, jnp.int32)
        shv = jnp.full((LANES,), shift, jnp.int32)
        bm = jnp.full((LANES,), binmask, jnp.int32)
        lane_off = lax.iota(jnp.int32, LANES) * HSTRIDE
        ones = jnp.ones((LANES,), jnp.int32)
        nwin_w = nbase + jnp.where(wid < nextra, 1, 0)

        def src(g):
            band = wid + g * NWORKERS
            return s_hbm.at[pl.ds(band * ROWS_PER_BAND, ROWS_PER_BAND), :]

        def start(g, buf, sem):
            pltpu.async_copy(src(g), buf, sem)

        def process(g, buf, sem):
            pltpu.make_async_copy(src(g), buf, sem).wait()

            for r in range(ROWS_PER_BAND):
                @plsc.parallel_loop(0, NPAD // LANES, step=1, unroll=UNROLL)
                def _vec(j, r=r):
                    x = buf[r, pl.ds(j * LANES, LANES)]
                    key = plsc.bitcast(x, jnp.int32)
                    b = lax.shift_right_logical(key, shv) & bm
                    idx = b + lane_off + (j % NCOPY) * (LANES * HSTRIDE)
                    if pmask == 0:
                        plsc.addupdate_scatter(hist, [idx], ones)
                    else:
                        sel = (key & pm) == pval
                        plsc.addupdate_scatter(hist, [idx], ones, mask=sel)

        start(0, buf0, sem0)

        def win_body(g, carry):
            even = (g % 2) == 0
            nxt = g + 1
            has_nxt = nxt < nwin_w

            @pl.when(jnp.logical_and(has_nxt, even))
            def _():
                start(nxt, buf1, sem1)

            @pl.when(jnp.logical_and(has_nxt, jnp.logical_not(even)))
            def _():
                start(nxt, buf0, sem0)

            @pl.when(even)
            def _():
                process(g, buf0, sem0)

            @pl.when(jnp.logical_not(even))
            def _():
                process(g, buf1, sem1)

            return carry

        lax.fori_loop(0, nwin_w, win_body, 0)
        pltpu.sync_copy(hist, out_hbm.at[pl.ds(wid * hwords, hwords)])

    return hist_kernel


_HIST_PASSES = [
    (20, 10, 0x00000000),
    (10, 10, 0xFFF00000),
    (0, 10, 0xFFFFFC00),
]
_HIST_KERNELS = None


def _get_hist_kernels():
    global _HIST_KERNELS
    if _HIST_KERNELS is None:
        _HIST_KERNELS = [_hist_pass(*p) for p in _HIST_PASSES]
    return _HIST_KERNELS


def _select_body(shift, hists_ref, carry_ref, out_ref):
    h = jnp.sum(hists_ref[...], axis=0, keepdims=True)
    iota = lax.broadcasted_iota(jnp.int32, (1, NBINS), 1)
    c = h
    sh = 1
    while sh < NBINS:
        r = pltpu.roll(c, sh, axis=1)
        c = c + jnp.where(iota >= sh, r, 0)
        sh *= 2
    tot = jnp.sum(h, axis=1, keepdims=True)
    cnt_gt = tot - c
    cnt_ge = cnt_gt + h
    krem = carry_ref[1]
    sel = (cnt_gt < krem) & (cnt_ge >= krem)
    bstar = jnp.max(jnp.where(sel, iota, -1), axis=(0, 1))
    cgt = jnp.max(jnp.where(sel, cnt_gt, -1), axis=(0, 1))
    out_ref[0] = carry_ref[0] + bstar * (1 << shift)
    out_ref[1] = krem - cgt
    out_ref[2] = 0
    out_ref[3] = 0


def _select_bin(shift, hists, carry):
    return pl.pallas_call(
        functools.partial(_select_body, shift),
        in_specs=[
            pl.BlockSpec((NWORKERS * NCOPY * LANES, NBINS), lambda: (0, 0)),
            pl.BlockSpec(memory_space=pltpu.SMEM),
        ],
        out_specs=pl.BlockSpec(memory_space=pltpu.SMEM),
        out_shape=jax.ShapeDtypeStruct((4,), jnp.int32),
    )(hists, carry)


def _mask_body(br, bc, s_ref, tk_ref, out_ref):
    i = pl.program_id(0)
    j = pl.program_id(1)
    s = s_ref[...]
    key = lax.bitcast_convert_type(s, jnp.int32)
    kept = jnp.where(key >= tk_ref[0], s, 0.0)
    row = i * br + lax.broadcasted_iota(jnp.int32, (br, bc), 0)
    col = j * bc + lax.broadcasted_iota(jnp.int32, (br, bc), 1)
    out_ref[...] = jnp.where(row == col, jnp.maximum(kept, 0.5), kept)


def _mask(S, tkey):
    br, bc = 256, 2048
    grid = (N // br + (N % br != 0), N // bc + (N % bc != 0))
    return pl.pallas_call(
        functools.partial(_mask_body, br, bc),
        grid=grid,
        in_specs=[
            pl.BlockSpec((br, bc), lambda i, j: (i, j)),
            pl.BlockSpec(memory_space=pltpu.SMEM),
        ],
        out_specs=pl.BlockSpec((br, bc), lambda i, j: (i, j)),
        out_shape=jax.ShapeDtypeStruct((N, N), jnp.float32),
    )(S, tkey)


def kernel(emb1, emb2, W1, b1, W2, b2):
    nv1, nv2 = _nodevecs(emb1, emb2, W1, b1, W2, b2)
    S = _scores(nv1, nv2)
    carry = jnp.array([0, NUM_EDGES, 0, 0], jnp.int32)
    for (shift, _w, _m), hk in zip(_HIST_PASSES, _get_hist_kernels()):
        pvalv = jnp.broadcast_to(carry[0], (LANES,)).astype(jnp.int32)
        hists = hk(S, pvalv)
        hists = hists.reshape(NWORKERS * NCOPY * LANES, HSTRIDE)[:, :NBINS]
        carry = _select_bin(shift, hists, carry)
    return _mask(S, carry)
